# baseline (speedup 1.0000x reference)
import functools

import jax
import jax.numpy as jnp
from jax import lax
from jax.experimental import pallas as pl
from jax.experimental.pallas import tpu as pltpu
from jax.experimental.pallas import tpu_sc as plsc

_NC = 2
_NS = 16
_NW = _NC * _NS

_CHUNK = 128


@functools.lru_cache(maxsize=None)
def _make_gather(V1, D, B):
  b_per_w = B // _NW
  n_chunks = b_per_w // _CHUNK
  mesh = plsc.VectorSubcoreMesh(core_axis_name="c", subcore_axis_name="s")

  @functools.partial(
      pl.kernel,
      mesh=mesh,
      compiler_params=pltpu.CompilerParams(use_tc_tiling_on_sc=False),
      out_type=jax.ShapeDtypeStruct((B, D), jnp.float32),
      scratch_types=[
          pltpu.VMEM((b_per_w,), jnp.int32),
          pltpu.VMEM((b_per_w, D), jnp.float32),
          pltpu.SemaphoreType.DMA,
      ],
  )
  def k(idx_hbm, table_hbm, out_hbm, idx_v, rows_v, sem):
    wid = lax.axis_index("s") * _NC + lax.axis_index("c")
    base = wid * b_per_w
    pltpu.sync_copy(idx_hbm.at[pl.ds(base, b_per_w)], idx_v)
    copies = []
    for c in range(n_chunks):
      copies.append(
          pltpu.async_copy(
              table_hbm.at[idx_v.at[pl.ds(c * _CHUNK, _CHUNK)]],
              rows_v.at[pl.ds(c * _CHUNK, _CHUNK), :],
              sem,
          )
      )
    for cp in copies:
      cp.wait()
    pltpu.sync_copy(rows_v, out_hbm.at[pl.ds(base, b_per_w)])

  return k


def kernel(idx, table, p_uncond):
  B = idx.shape[0]
  V1, D = table.shape
  rkey = jax.random.fold_in(jax.random.key(0), 1)
  mask = jax.random.uniform(rkey, (B,)) < p_uncond
  idx = jnp.where(mask, V1 - 1, idx).astype(jnp.int32)
  out = _make_gather(V1, D, B)(idx, table)
  return out[:, None, :]


# speedup vs baseline: 1.4598x; 1.4598x over previous
import functools

import jax
import jax.numpy as jnp
from jax import lax
from jax.experimental import pallas as pl
from jax.experimental.pallas import tpu as pltpu
from jax.experimental.pallas import tpu_sc as plsc

_NC = 2
_NS = 16
_NW = _NC * _NS


@functools.lru_cache(maxsize=None)
def _make_gather(V1, D, B):
  b_per_w = B // _NW
  mesh = plsc.VectorSubcoreMesh(core_axis_name="c", subcore_axis_name="s")

  @functools.partial(
      pl.kernel,
      mesh=mesh,
      out_type=jax.ShapeDtypeStruct((B, D), jnp.float32),
      scratch_types=[
          pltpu.VMEM((b_per_w,), jnp.int32),
          pltpu.VMEM((b_per_w, D), jnp.float32),
          pltpu.SemaphoreType.DMA,
      ],
  )
  def k(idx_hbm, table_hbm, out_hbm, idx_v, rows_v, sem):
    wid = lax.axis_index("s") * _NC + lax.axis_index("c")
    base = wid * b_per_w
    pltpu.sync_copy(idx_hbm.at[pl.ds(base, b_per_w)], idx_v)

    @pl.loop(0, b_per_w // 16)
    def _(g):
      vec = idx_v[pl.ds(g * 16, 16)]
      for j in range(16):
        pltpu.async_copy(
            table_hbm.at[pl.ds(vec[j], 1), :],
            rows_v.at[pl.ds(g * 16 + j, 1), :],
            sem,
        )

    pltpu.make_async_copy(
        table_hbm.at[pl.ds(0, b_per_w), :], rows_v, sem
    ).wait()
    pltpu.sync_copy(rows_v, out_hbm.at[pl.ds(base, b_per_w)])

  return k


def kernel(idx, table, p_uncond):
  B = idx.shape[0]
  V1, D = table.shape
  rkey = jax.random.fold_in(jax.random.key(0), 1)
  mask = jax.random.uniform(rkey, (B,)) < p_uncond
  idx = jnp.where(mask, V1 - 1, idx).astype(jnp.int32)
  out = _make_gather(V1, D, B)(idx, table)
  return out[:, None, :]
